# Initial kernel scaffold; baseline (speedup 1.0000x reference)
import jax
import jax.numpy as jnp
from jax.experimental import pallas as pl


def kernel(x, edge_index, W_l1, b_l1, W_r1, W_l2, b_l2, W_r2, W_l3, b_l3, W_r3):
    raise NotImplementedError("write your pallas kernel here")


# speedup vs baseline: 3.3964x; 3.3964x over previous
import jax
import jax.numpy as jnp
from jax import lax
from jax.experimental import pallas as pl
from jax.experimental.pallas import tpu as pltpu
from jax.experimental.pallas import tpu_sc as plsc

N = 10000
D = 128
E = 320000
NC = 2
NS = 16
CH = 128
KCH = 157
EPAD = NS * KCH * CH
NH = 5120
NHB = NH + 8
RPT = NH // NS
_COPIES = ((0, 128), (128, 128), (256, 64))

_mesh = plsc.VectorSubcoreMesh(core_axis_name="c", subcore_axis_name="s")


def _make_agg():
    scratch = [
        pltpu.VMEM((KCH, CH), jnp.int32),
        pltpu.VMEM((KCH, CH), jnp.int32),
        pltpu.VMEM((CH, D), jnp.float32),
        pltpu.VMEM((128, D), jnp.float32),
        pltpu.VMEM_SHARED((NHB, D), jnp.float32),
        pltpu.SemaphoreType.DMA,
    ]

    def body(h_hbm, src_hbm, ldst_hbm, agg_out,
             src_v, ldst_v, rows_v, zrow_v, agg_sh, sem):
        c = lax.axis_index("c")
        s = lax.axis_index("s")

        def zf(i, carry):
            r = i // (D // 16)
            col = (i % (D // 16)) * 16
            zrow_v[r, pl.ds(col, 16)] = jnp.zeros((16,), jnp.float32)
            return carry
        lax.fori_loop(0, 128 * (D // 16), zf, 0)

        for off, nr in _COPIES:
            pltpu.sync_copy(zrow_v.at[pl.ds(0, nr)],
                            agg_sh.at[pl.ds(s * RPT + off, nr)])

        @pl.when(s == 0)
        def _zero_trash():
            pltpu.sync_copy(zrow_v.at[pl.ds(0, 8)], agg_sh.at[pl.ds(NH, 8)])

        pltpu.sync_copy(src_hbm.at[s], src_v)
        pltpu.sync_copy(ldst_hbm.at[c, s], ldst_v)
        plsc.subcore_barrier()

        def step(j, carry):
            pltpu.async_copy(h_hbm.at[src_v.at[j]], rows_v, sem).wait()
            pltpu.sync_copy(rows_v, agg_sh.at[ldst_v.at[j]], add=True)
            return carry
        lax.fori_loop(0, KCH, step, 0)
        plsc.subcore_barrier()

        for off, nr in _COPIES:
            o = s * RPT + off
            pltpu.sync_copy(agg_sh.at[pl.ds(o, nr)],
                            agg_out.at[c, pl.ds(o, nr)])

    return pl.kernel(
        body,
        out_type=[jax.ShapeDtypeStruct((NC, NH, D), jnp.float32)],
        mesh=_mesh, scratch_types=scratch)


def _make_cnt():
    scratch = [
        pltpu.VMEM((KCH, CH), jnp.int32),
        pltpu.VMEM((CH, D), jnp.float32),
        pltpu.VMEM((128, D), jnp.float32),
        pltpu.VMEM_SHARED((NHB, D), jnp.float32),
    ]

    def body(ldst_hbm, cnt_out, ldst_v, ones_v, zrow_v, cnt_sh):
        c = lax.axis_index("c")
        s = lax.axis_index("s")

        def of(i, carry):
            r = i // (D // 16)
            col = (i % (D // 16)) * 16
            ones_v[r, pl.ds(col, 16)] = jnp.ones((16,), jnp.float32)
            return carry
        lax.fori_loop(0, CH * (D // 16), of, 0)

        def zf(i, carry):
            r = i // (D // 16)
            col = (i % (D // 16)) * 16
            zrow_v[r, pl.ds(col, 16)] = jnp.zeros((16,), jnp.float32)
            return carry
        lax.fori_loop(0, 128 * (D // 16), zf, 0)

        for off, nr in _COPIES:
            pltpu.sync_copy(zrow_v.at[pl.ds(0, nr)],
                            cnt_sh.at[pl.ds(s * RPT + off, nr)])

        @pl.when(s == 0)
        def _zero_trash():
            pltpu.sync_copy(zrow_v.at[pl.ds(0, 8)], cnt_sh.at[pl.ds(NH, 8)])

        pltpu.sync_copy(ldst_hbm.at[c, s], ldst_v)
        plsc.subcore_barrier()

        def step(j, carry):
            pltpu.sync_copy(ones_v, cnt_sh.at[ldst_v.at[j]], add=True)
            return carry
        lax.fori_loop(0, KCH, step, 0)
        plsc.subcore_barrier()

        for off, nr in _COPIES:
            o = s * RPT + off
            pltpu.sync_copy(cnt_sh.at[pl.ds(o, nr)],
                            cnt_out.at[c, pl.ds(o, nr)])

    return pl.kernel(
        body,
        out_type=[jax.ShapeDtypeStruct((NC, NH, D), jnp.float32)],
        mesh=_mesh, scratch_types=scratch)


_agg_k = _make_agg()
_cnt_k = _make_cnt()

R = 1000


def _mm_body(a_ref, cnt_ref, h_ref, wl_ref, wr_ref, b_ref, fl_ref, o_ref):
    inv = 1.0 / jnp.maximum(cnt_ref[:, 0:1], 1.0)
    mean = a_ref[...] * inv
    acc = jnp.dot(mean, wl_ref[...], preferred_element_type=jnp.float32)
    acc = acc + jnp.dot(h_ref[...], wr_ref[...],
                        preferred_element_type=jnp.float32)
    acc = acc + b_ref[...]
    o_ref[...] = jnp.where(fl_ref[...] > 0.0, jnp.maximum(acc, 0.0), acc)


_mm_k = pl.pallas_call(
    _mm_body,
    grid=(N // R,),
    in_specs=[
        pl.BlockSpec((R, D), lambda i: (i, 0)),
        pl.BlockSpec((R, D), lambda i: (i, 0)),
        pl.BlockSpec((R, D), lambda i: (i, 0)),
        pl.BlockSpec((D, D), lambda i: (0, 0)),
        pl.BlockSpec((D, D), lambda i: (0, 0)),
        pl.BlockSpec((1, D), lambda i: (0, 0)),
        pl.BlockSpec((1, D), lambda i: (0, 0)),
    ],
    out_specs=pl.BlockSpec((R, D), lambda i: (i, 0)),
    out_shape=jax.ShapeDtypeStruct((N, D), jnp.float32),
)


@jax.jit
def kernel(x, edge_index, W_l1, b_l1, W_r1, W_l2, b_l2, W_r2,
           W_l3, b_l3, W_r3):
    pad = EPAD - E
    src = jnp.concatenate([edge_index[0],
                           jnp.zeros((pad,), jnp.int32)]).reshape(NS, KCH, CH)
    dst = jnp.concatenate([edge_index[1], jnp.full((pad,), -1, jnp.int32)])
    base = jnp.arange(NC, dtype=jnp.int32)[:, None] * NH
    local = dst[None, :] - base
    ldst = jnp.where((local >= 0) & (local < NH), local, NH)
    ldst = ldst.reshape(NC, NS, KCH, CH)

    cnt, = _cnt_k(ldst)
    cnt = cnt.reshape(NC * NH, D)

    Wl = jnp.stack([W_l1, W_l2, W_l3])
    Wr = jnp.stack([W_r1, W_r2, W_r3])
    B = jnp.stack([b_l1, b_l2, b_l3]).reshape(3, 1, D)
    FL = jnp.array([1.0, 1.0, 0.0], jnp.float32)[:, None, None] \
        * jnp.ones((1, 1, D), jnp.float32)

    def step(h, lyr):
        wl, wr, bb, fl = lyr
        agg, = _agg_k(h, src, ldst)
        h2 = _mm_k(agg.reshape(NC * NH, D), cnt, h, wl, wr, bb, fl)
        return h2, None

    h, _ = lax.scan(step, x, (Wl, Wr, B, FL))
    return h
